# Initial kernel scaffold; baseline (speedup 1.0000x reference)
import jax
import jax.numpy as jnp
from jax.experimental import pallas as pl


def kernel(ids, table):
    raise NotImplementedError("write your pallas kernel here")


# speedup vs baseline: 3.0156x; 3.0156x over previous
import functools

import jax
import jax.numpy as jnp
from jax import lax
from jax.experimental import pallas as pl
from jax.experimental.pallas import tpu as pltpu
from jax.experimental.pallas import tpu_sc as plsc

EMBED_DIM = 64
_LANES = 16

_mesh = plsc.VectorSubcoreMesh(core_axis_name="c", subcore_axis_name="s")


@functools.partial(
    pl.kernel,
    mesh=_mesh,
    out_type=jax.ShapeDtypeStruct((EMBED_DIM,), jnp.float32),
    scratch_types=[
        pltpu.VMEM((_LANES,), jnp.int32),
        pltpu.VMEM((EMBED_DIM,), jnp.float32),
    ],
)
def _sc_lookup(ids_hbm, table_hbm, out_hbm, idx_v, row_v):
    c = lax.axis_index("c")
    s = lax.axis_index("s")

    @pl.when(jnp.logical_and(c == 0, s == 0))
    def _():
        pltpu.sync_copy(ids_hbm.at[pl.ds(0, _LANES)], idx_v)
        idx0 = idx_v[...][0]
        off = pl.multiple_of(idx0 * EMBED_DIM, 8)
        pltpu.sync_copy(table_hbm.at[pl.ds(off, EMBED_DIM)], row_v)
        pltpu.sync_copy(row_v, out_hbm)


def kernel(ids, table):
    ids_flat = ids.reshape(-1).astype(jnp.int32)
    table_flat = table.reshape(-1)
    return _sc_lookup(ids_flat, table_flat)


# speedup vs baseline: 5.1897x; 1.7210x over previous
import functools

import jax
import jax.numpy as jnp
from jax import lax
from jax.experimental import pallas as pl
from jax.experimental.pallas import tpu as pltpu
from jax.experimental.pallas import tpu_sc as plsc

EMBED_DIM = 64
_LANES = 16

_mesh = plsc.VectorSubcoreMesh(core_axis_name="c", subcore_axis_name="s")


@functools.partial(
    pl.kernel,
    mesh=_mesh,
    out_type=jax.ShapeDtypeStruct((EMBED_DIM,), jnp.float32),
    scratch_types=[
        pltpu.VMEM((_LANES,), jnp.int32),
        pltpu.VMEM((1, EMBED_DIM), jnp.float32),
    ],
)
def _sc_lookup(ids_hbm, table_hbm, out_hbm, idx_v, row_v):
    c = lax.axis_index("c")
    s = lax.axis_index("s")

    @pl.when(jnp.logical_and(c == 0, s == 0))
    def _():
        pltpu.sync_copy(ids_hbm.at[0, pl.ds(0, _LANES)], idx_v)
        idx0 = idx_v[...][0]
        pltpu.sync_copy(table_hbm.at[pl.ds(idx0, 1), :], row_v)
        pltpu.sync_copy(row_v.at[0], out_hbm)


def kernel(ids, table):
    return _sc_lookup(ids.astype(jnp.int32), table)
